# Initial kernel scaffold; baseline (speedup 1.0000x reference)
import functools

import jax
import jax.numpy as jnp
from jax import lax
from jax.experimental import pallas as pl
from jax.experimental.pallas import tpu as pltpu
from jax.experimental.pallas import tpu_sc as plsc

N = 10000
E = 320000
D = 128
H = 128
ED = 16

NC = 2
NS = 16
NW = NC * NS
K = 80
EPT = E // NW
CHUNKS = EPT // K
RPT = N // NS
LANES = 16

_DOT = functools.partial(
    lax.dot_general,
    dimension_numbers=(((1,), (0,)), ((), ())),
    preferred_element_type=jnp.float32,
    precision=lax.Precision.HIGHEST,
)


def _pre_body(x_ref, wi_ref, wj_ref, a_ref, b_ref):
    xb = x_ref[...]
    a_ref[...] = _DOT(xb, wi_ref[...])
    b_ref[...] = _DOT(xb, wj_ref[...])


def _cpre_body(ea_ref, we0_ref, b0_ref, we1_ref, b1_ref, c0_ref, c1_ref):
    ea = ea_ref[...]
    c0_ref[...] = _DOT(ea, we0_ref[...]) + b0_ref[...]
    c1_ref[...] = _DOT(ea, we1_ref[...]) + b1_ref[...]


def _upd_fused_body(x_ref, a0_ref, a1_ref, wux_ref, wua_ref, bu_ref,
                    wi_ref, wj_ref, h_ref, a_ref, b_ref):
    agg = a0_ref[...] + a1_ref[...]
    h = _DOT(x_ref[...], wux_ref[...]) + _DOT(agg, wua_ref[...]) + bu_ref[...]
    h = jnp.maximum(h, 0.0)
    h_ref[...] = h
    a_ref[...] = _DOT(h, wi_ref[...])
    b_ref[...] = _DOT(h, wj_ref[...])


def _upd_body(x_ref, a0_ref, a1_ref, wux_ref, wua_ref, bu_ref, h_ref):
    agg = a0_ref[...] + a1_ref[...]
    h = _DOT(x_ref[...], wux_ref[...]) + _DOT(agg, wua_ref[...]) + bu_ref[...]
    h_ref[...] = jnp.maximum(h, 0.0)


_BN = 1000
_BE = 4000


def _node_spec(shape):
    return pl.BlockSpec((_BN,) + shape[1:], lambda i: (i,) + (0,) * (len(shape) - 1))


def _full_spec(shape):
    return pl.BlockSpec(shape, lambda i: (0,) * len(shape))


def _tc_pre(x, wi, wj):
    return pl.pallas_call(
        _pre_body,
        grid=(N // _BN,),
        in_specs=[_node_spec((N, D)), _full_spec((D, H)), _full_spec((D, H))],
        out_specs=[_node_spec((N, H)), _node_spec((N, H))],
        out_shape=[jax.ShapeDtypeStruct((N, H), jnp.float32)] * 2,
    )(x, wi, wj)


def _tc_cpre(ea, we0, b0, we1, b1):
    espec = pl.BlockSpec((_BE, ED), lambda i: (i, 0))
    ospec = pl.BlockSpec((_BE, H), lambda i: (i, 0))
    return pl.pallas_call(
        _cpre_body,
        grid=(E // _BE,),
        in_specs=[espec, _full_spec((ED, H)), _full_spec((1, H)),
                  _full_spec((ED, H)), _full_spec((1, H))],
        out_specs=[ospec, ospec],
        out_shape=[jax.ShapeDtypeStruct((E, H), jnp.float32)] * 2,
    )(ea, we0, b0, we1, b1)


def _tc_update_fused(x, aggs, wux, wua, bu, wi, wj):
    a0spec = pl.BlockSpec((_BN, H), lambda i: (i, 0))
    a1spec = pl.BlockSpec((_BN, H), lambda i: (i + N // _BN, 0))
    return pl.pallas_call(
        _upd_fused_body,
        grid=(N // _BN,),
        in_specs=[_node_spec((N, D)), a0spec, a1spec,
                  _full_spec((D, H)), _full_spec((H, H)), _full_spec((1, H)),
                  _full_spec((H, H)), _full_spec((H, H))],
        out_specs=[_node_spec((N, H))] * 3,
        out_shape=[jax.ShapeDtypeStruct((N, H), jnp.float32)] * 3,
    )(x, aggs, aggs, wux, wua, bu, wi, wj)


def _tc_update(x, aggs, wux, wua, bu):
    a0spec = pl.BlockSpec((_BN, H), lambda i: (i, 0))
    a1spec = pl.BlockSpec((_BN, H), lambda i: (i + N // _BN, 0))
    return pl.pallas_call(
        _upd_body,
        grid=(N // _BN,),
        in_specs=[_node_spec((N, H)), a0spec, a1spec,
                  _full_spec((H, H)), _full_spec((H, H)), _full_spec((1, H))],
        out_specs=_node_spec((N, H)),
        out_shape=jax.ShapeDtypeStruct((N, H), jnp.float32),
    )(x, aggs, aggs, wux, wua, bu)


def _sc_edge_body(a_hbm, b_hbm, c_hbm, src_hbm, dst_hbm, out_hbm,
                  shared, dst_v, src_v, buf_a, buf_b, buf_c, sem_a, sem_b):
    c = lax.axis_index("c")
    s = lax.axis_index("s")
    g = c * NS + s

    zero = jnp.zeros((LANES,), jnp.float32)

    def _zrow(r, _):
        for j in range(H // LANES):
            buf_a[r, pl.ds(j * LANES, LANES)] = zero
        return 0
    lax.fori_loop(0, K, _zrow, 0, unroll=False)

    rbase = s * RPT
    nfull = RPT // K
    rem = RPT - nfull * K
    for j in range(nfull):
        pltpu.sync_copy(buf_a, shared.at[pl.ds(rbase + j * K, K)])
    if rem:
        pltpu.sync_copy(buf_a.at[pl.ds(0, rem)],
                        shared.at[pl.ds(rbase + nfull * K, rem)])

    plsc.subcore_barrier()

    def _chunk(ch, _):
        eoff = pl.multiple_of(g * EPT + ch * K, K)
        pltpu.sync_copy(dst_hbm.at[pl.ds(eoff, K)], dst_v)
        pltpu.sync_copy(src_hbm.at[pl.ds(eoff, K)], src_v)
        cp_a = pltpu.async_copy(a_hbm.at[dst_v], buf_a, sem_a)
        cp_b = pltpu.async_copy(b_hbm.at[src_v], buf_b, sem_b)
        pltpu.sync_copy(c_hbm.at[pl.ds(eoff, K)], buf_c)
        cp_a.wait()
        cp_b.wait()

        def _row(r, _):
            for j in range(H // LANES):
                sl = pl.ds(j * LANES, LANES)
                v = buf_a[r, sl] + buf_b[r, sl] + buf_c[r, sl]
                buf_a[r, sl] = jnp.maximum(v, 0.0)
            return 0
        lax.fori_loop(0, K, _row, 0, unroll=False)

        pltpu.sync_copy(buf_a, shared.at[dst_v], add=True)
        return 0

    lax.fori_loop(0, CHUNKS, _chunk, 0, unroll=False)

    plsc.subcore_barrier()

    obase = c * N + rbase
    for j in range(nfull):
        pltpu.sync_copy(shared.at[pl.ds(rbase + j * K, K)], buf_a)
        pltpu.sync_copy(buf_a, out_hbm.at[pl.ds(obase + j * K, K)])
    if rem:
        pltpu.sync_copy(shared.at[pl.ds(rbase + nfull * K, rem)],
                        buf_a.at[pl.ds(0, rem)])
        pltpu.sync_copy(buf_a.at[pl.ds(0, rem)],
                        out_hbm.at[pl.ds(obase + nfull * K, rem)])


_sc_edge = pl.kernel(
    _sc_edge_body,
    out_type=jax.ShapeDtypeStruct((NC * N, H), jnp.float32),
    mesh=plsc.VectorSubcoreMesh(core_axis_name="c", subcore_axis_name="s"),
    scratch_types=[
        pltpu.VMEM_SHARED((N, H), jnp.float32),
        pltpu.VMEM((K,), jnp.int32),
        pltpu.VMEM((K,), jnp.int32),
        pltpu.VMEM((K, H), jnp.float32),
        pltpu.VMEM((K, H), jnp.float32),
        pltpu.VMEM((K, H), jnp.float32),
        pltpu.SemaphoreType.DMA,
        pltpu.SemaphoreType.DMA,
    ],
)


@jax.jit
def kernel(x, edge_index, edge_attr, W_msg0, b_msg0, W_upd0, b_upd0,
           W_msg1, b_msg1, W_upd1, b_upd1):
    src = edge_index[0]
    dst = edge_index[1]

    b0 = b_msg0.reshape(1, H)
    b1 = b_msg1.reshape(1, H)
    bu0 = b_upd0.reshape(1, H)
    bu1 = b_upd1.reshape(1, H)

    c0, c1 = _tc_cpre(edge_attr, W_msg0[2 * D:], b0, W_msg1[2 * H:], b1)

    a0, bmat0 = _tc_pre(x, W_msg0[:D], W_msg0[D:2 * D])
    aggs0 = _sc_edge(a0, bmat0, c0, src, dst)
    h, a1, bmat1 = _tc_update_fused(
        x, aggs0, W_upd0[:D], W_upd0[D:], bu0,
        W_msg1[:H], W_msg1[H:2 * H])

    aggs1 = _sc_edge(a1, bmat1, c1, src, dst)
    out = _tc_update(h, aggs1, W_upd1[:H], W_upd1[H:], bu1)
    return out


# speedup vs baseline: 3.4065x; 3.4065x over previous
import functools

import jax
import jax.numpy as jnp
from jax import lax
from jax.experimental import pallas as pl
from jax.experimental.pallas import tpu as pltpu
from jax.experimental.pallas import tpu_sc as plsc

N = 10000
E = 320000
D = 128
H = 128
ED = 16

NC = 2
NS = 16
NW = NC * NS
K = 80
EPT = E // NW
CHUNKS = EPT // K
NP = 10240
RPT = NP // NS
LANES = 16

_DOT = functools.partial(
    lax.dot_general,
    dimension_numbers=(((1,), (0,)), ((), ())),
    preferred_element_type=jnp.float32,
    precision=lax.Precision.HIGHEST,
)


def _pre_body(x_ref, wi_ref, wj_ref, a_ref, b_ref):
    xb = x_ref[...]
    a_ref[...] = _DOT(xb, wi_ref[...])
    b_ref[...] = _DOT(xb, wj_ref[...])


def _cpre_body(ea_ref, we0_ref, b0_ref, we1_ref, b1_ref, c0_ref, c1_ref):
    ea = ea_ref[...]
    c0_ref[...] = _DOT(ea, we0_ref[...]) + b0_ref[...]
    c1_ref[...] = _DOT(ea, we1_ref[...]) + b1_ref[...]


def _upd_fused_body(x_ref, a0_ref, a1_ref, wux_ref, wua_ref, bu_ref,
                    wi_ref, wj_ref, h_ref, a_ref, b_ref):
    agg = a0_ref[0] + a1_ref[0]
    h = _DOT(x_ref[...], wux_ref[...]) + _DOT(agg, wua_ref[...]) + bu_ref[...]
    h = jnp.maximum(h, 0.0)
    h_ref[...] = h
    a_ref[...] = _DOT(h, wi_ref[...])
    b_ref[...] = _DOT(h, wj_ref[...])


def _upd_body(x_ref, a0_ref, a1_ref, wux_ref, wua_ref, bu_ref, h_ref):
    agg = a0_ref[0] + a1_ref[0]
    h = _DOT(x_ref[...], wux_ref[...]) + _DOT(agg, wua_ref[...]) + bu_ref[...]
    h_ref[...] = jnp.maximum(h, 0.0)


_BN = 1000
_BE = 4000


def _node_spec(shape):
    return pl.BlockSpec((_BN,) + shape[1:], lambda i: (i,) + (0,) * (len(shape) - 1))


def _full_spec(shape):
    return pl.BlockSpec(shape, lambda i: (0,) * len(shape))


def _tc_pre(x, wi, wj):
    return pl.pallas_call(
        _pre_body,
        grid=(N // _BN,),
        in_specs=[_node_spec((N, D)), _full_spec((D, H)), _full_spec((D, H))],
        out_specs=[_node_spec((N, H)), _node_spec((N, H))],
        out_shape=[jax.ShapeDtypeStruct((N, H), jnp.float32)] * 2,
    )(x, wi, wj)


def _tc_cpre(ea, we0, b0, we1, b1):
    espec = pl.BlockSpec((_BE, ED), lambda i: (i, 0))
    ospec = pl.BlockSpec((_BE, H), lambda i: (i, 0))
    return pl.pallas_call(
        _cpre_body,
        grid=(E // _BE,),
        in_specs=[espec, _full_spec((ED, H)), _full_spec((1, H)),
                  _full_spec((ED, H)), _full_spec((1, H))],
        out_specs=[ospec, ospec],
        out_shape=[jax.ShapeDtypeStruct((E, H), jnp.float32)] * 2,
    )(ea, we0, b0, we1, b1)


_A0SPEC = pl.BlockSpec((1, _BN, H), lambda i: (0, i, 0))
_A1SPEC = pl.BlockSpec((1, _BN, H), lambda i: (1, i, 0))


def _tc_update_fused(x, aggs, wux, wua, bu, wi, wj):
    return pl.pallas_call(
        _upd_fused_body,
        grid=(N // _BN,),
        in_specs=[_node_spec((N, D)), _A0SPEC, _A1SPEC,
                  _full_spec((D, H)), _full_spec((H, H)), _full_spec((1, H)),
                  _full_spec((H, H)), _full_spec((H, H))],
        out_specs=[_node_spec((N, H))] * 3,
        out_shape=[jax.ShapeDtypeStruct((N, H), jnp.float32)] * 3,
    )(x, aggs, aggs, wux, wua, bu, wi, wj)


def _tc_update(x, aggs, wux, wua, bu):
    return pl.pallas_call(
        _upd_body,
        grid=(N // _BN,),
        in_specs=[_node_spec((N, H)), _A0SPEC, _A1SPEC,
                  _full_spec((H, H)), _full_spec((H, H)), _full_spec((1, H))],
        out_specs=_node_spec((N, H)),
        out_shape=jax.ShapeDtypeStruct((N, H), jnp.float32),
    )(x, aggs, aggs, wux, wua, bu)


def _sc_edge_body(a_hbm, b_hbm, c_hbm, src_hbm, dst_hbm, out_hbm,
                  shared, dst_v, src_v, buf_a, buf_b, buf_c, sem_a, sem_b):
    c = lax.axis_index("c")
    s = lax.axis_index("s")
    g = c * NS + s

    zero = jnp.zeros((LANES,), jnp.float32)

    def _zrow(r, _):
        for j in range(H // LANES):
            buf_a[r, pl.ds(j * LANES, LANES)] = zero
        return 0
    lax.fori_loop(0, K, _zrow, 0, unroll=False)

    rbase = pl.multiple_of(s * RPT, K)
    for j in range(RPT // K):
        pltpu.sync_copy(buf_a, shared.at[pl.ds(rbase + j * K, K)])

    plsc.subcore_barrier()

    def _chunk(ch, _):
        eoff = pl.multiple_of(g * EPT + ch * K, K)
        pltpu.sync_copy(dst_hbm.at[pl.ds(eoff, K)], dst_v)
        pltpu.sync_copy(src_hbm.at[pl.ds(eoff, K)], src_v)
        cp_a = pltpu.async_copy(a_hbm.at[dst_v], buf_a, sem_a)
        cp_b = pltpu.async_copy(b_hbm.at[src_v], buf_b, sem_b)
        pltpu.sync_copy(c_hbm.at[pl.ds(eoff, K)], buf_c)
        cp_a.wait()
        cp_b.wait()

        def _row(r, _):
            for j in range(H // LANES):
                sl = pl.ds(j * LANES, LANES)
                v = buf_a[r, sl] + buf_b[r, sl] + buf_c[r, sl]
                buf_a[r, sl] = jnp.maximum(v, 0.0)
            return 0
        lax.fori_loop(0, K, _row, 0, unroll=False)

        pltpu.sync_copy(buf_a, shared.at[dst_v], add=True)
        return 0

    lax.fori_loop(0, CHUNKS, _chunk, 0, unroll=False)

    plsc.subcore_barrier()

    obase = pl.multiple_of(c * NP + rbase, K)
    for j in range(RPT // K):
        pltpu.sync_copy(shared.at[pl.ds(rbase + j * K, K)], buf_a)
        pltpu.sync_copy(buf_a, out_hbm.at[pl.ds(obase + j * K, K)])


@functools.cache
def _sc_edge_kernel():
    return pl.kernel(
        _sc_edge_body,
        out_type=jax.ShapeDtypeStruct((NC * NP, H), jnp.float32),
        mesh=plsc.VectorSubcoreMesh(core_axis_name="c", subcore_axis_name="s"),
        scratch_types=[
            pltpu.VMEM_SHARED((NP, H), jnp.float32),
            pltpu.VMEM((K,), jnp.int32),
            pltpu.VMEM((K,), jnp.int32),
            pltpu.VMEM((K, H), jnp.float32),
            pltpu.VMEM((K, H), jnp.float32),
            pltpu.VMEM((K, H), jnp.float32),
            pltpu.SemaphoreType.DMA,
            pltpu.SemaphoreType.DMA,
        ],
    )


def _sc_edge(a, b, c, src, dst):
    return _sc_edge_kernel()(a, b, c, src, dst)


@jax.jit
def kernel(x, edge_index, edge_attr, W_msg0, b_msg0, W_upd0, b_upd0,
           W_msg1, b_msg1, W_upd1, b_upd1):
    src = edge_index[0]
    dst = edge_index[1]

    b0 = b_msg0.reshape(1, H)
    b1 = b_msg1.reshape(1, H)
    bu0 = b_upd0.reshape(1, H)
    bu1 = b_upd1.reshape(1, H)

    c0, c1 = _tc_cpre(edge_attr, W_msg0[2 * D:], b0, W_msg1[2 * H:], b1)

    a0, bmat0 = _tc_pre(x, W_msg0[:D], W_msg0[D:2 * D])
    aggs0 = _sc_edge(a0, bmat0, c0, src, dst).reshape(NC, NP, H)
    h, a1, bmat1 = _tc_update_fused(
        x, aggs0, W_upd0[:D], W_upd0[D:], bu0,
        W_msg1[:H], W_msg1[H:2 * H])

    aggs1 = _sc_edge(a1, bmat1, c1, src, dst).reshape(NC, NP, H)
    out = _tc_update(h, aggs1, W_upd1[:H], W_upd1[H:], bu1)
    return out


# speedup vs baseline: 5.2306x; 1.5355x over previous
import functools

import jax
import jax.numpy as jnp
from jax import lax
from jax.experimental import pallas as pl
from jax.experimental.pallas import tpu as pltpu
from jax.experimental.pallas import tpu_sc as plsc

N = 10000
E = 320000
D = 128
H = 128
ED = 16

NC = 2
NS = 16
NW = NC * NS
K = 40
EPT = E // NW
CHUNKS = EPT // K
IBLK = 25
NBLK = CHUNKS // IBLK
NP = 10240
RPT = NP // NS
LANES = 16

_DOT = functools.partial(
    lax.dot_general,
    dimension_numbers=(((1,), (0,)), ((), ())),
    preferred_element_type=jnp.float32,
    precision=lax.Precision.HIGHEST,
)


def _pre_body(x_ref, wi_ref, wj_ref, a_ref, b_ref):
    xb = x_ref[...]
    a_ref[...] = _DOT(xb, wi_ref[...])
    b_ref[...] = _DOT(xb, wj_ref[...])


def _cpre_body(ea_ref, we0_ref, b0_ref, we1_ref, b1_ref, c0_ref, c1_ref):
    ea = ea_ref[...]
    c0_ref[...] = _DOT(ea, we0_ref[...]) + b0_ref[...]
    c1_ref[...] = _DOT(ea, we1_ref[...]) + b1_ref[...]


def _upd_fused_body(x_ref, a0_ref, a1_ref, wux_ref, wua_ref, bu_ref,
                    wi_ref, wj_ref, h_ref, a_ref, b_ref):
    agg = a0_ref[0] + a1_ref[0]
    h = _DOT(x_ref[...], wux_ref[...]) + _DOT(agg, wua_ref[...]) + bu_ref[...]
    h = jnp.maximum(h, 0.0)
    h_ref[...] = h
    a_ref[...] = _DOT(h, wi_ref[...])
    b_ref[...] = _DOT(h, wj_ref[...])


def _upd_body(x_ref, a0_ref, a1_ref, wux_ref, wua_ref, bu_ref, h_ref):
    agg = a0_ref[0] + a1_ref[0]
    h = _DOT(x_ref[...], wux_ref[...]) + _DOT(agg, wua_ref[...]) + bu_ref[...]
    h_ref[...] = jnp.maximum(h, 0.0)


_BN = 1000
_BE = 4000


def _node_spec(shape):
    return pl.BlockSpec((_BN,) + shape[1:], lambda i: (i,) + (0,) * (len(shape) - 1))


def _full_spec(shape):
    return pl.BlockSpec(shape, lambda i: (0,) * len(shape))


def _tc_pre(x, wi, wj):
    return pl.pallas_call(
        _pre_body,
        grid=(N // _BN,),
        in_specs=[_node_spec((N, D)), _full_spec((D, H)), _full_spec((D, H))],
        out_specs=[_node_spec((N, H)), _node_spec((N, H))],
        out_shape=[jax.ShapeDtypeStruct((N, H), jnp.float32)] * 2,
    )(x, wi, wj)


def _tc_cpre(ea, we0, b0, we1, b1):
    espec = pl.BlockSpec((_BE, ED), lambda i: (i, 0))
    ospec = pl.BlockSpec((_BE, H), lambda i: (i, 0))
    return pl.pallas_call(
        _cpre_body,
        grid=(E // _BE,),
        in_specs=[espec, _full_spec((ED, H)), _full_spec((1, H)),
                  _full_spec((ED, H)), _full_spec((1, H))],
        out_specs=[ospec, ospec],
        out_shape=[jax.ShapeDtypeStruct((E, H), jnp.float32)] * 2,
    )(ea, we0, b0, we1, b1)


_A0SPEC = pl.BlockSpec((1, _BN, H), lambda i: (0, i, 0))
_A1SPEC = pl.BlockSpec((1, _BN, H), lambda i: (1, i, 0))


def _tc_update_fused(x, aggs, wux, wua, bu, wi, wj):
    return pl.pallas_call(
        _upd_fused_body,
        grid=(N // _BN,),
        in_specs=[_node_spec((N, D)), _A0SPEC, _A1SPEC,
                  _full_spec((D, H)), _full_spec((H, H)), _full_spec((1, H)),
                  _full_spec((H, H)), _full_spec((H, H))],
        out_specs=[_node_spec((N, H))] * 3,
        out_shape=[jax.ShapeDtypeStruct((N, H), jnp.float32)] * 3,
    )(x, aggs, aggs, wux, wua, bu, wi, wj)


def _tc_update(x, aggs, wux, wua, bu):
    return pl.pallas_call(
        _upd_body,
        grid=(N // _BN,),
        in_specs=[_node_spec((N, H)), _A0SPEC, _A1SPEC,
                  _full_spec((H, H)), _full_spec((H, H)), _full_spec((1, H))],
        out_specs=_node_spec((N, H)),
        out_shape=jax.ShapeDtypeStruct((N, H), jnp.float32),
    )(x, aggs, aggs, wux, wua, bu)


def _sc_edge_body(a_hbm, b_hbm, c_hbm, src_hbm, dst_hbm, out_hbm,
                  shared, idx_d0, idx_d1, idx_s0, idx_s1,
                  buf_a0, buf_a1, buf_b0, buf_b1, buf_c0, buf_c1,
                  sem_a0, sem_a1, sem_b0, sem_b1, sem_c0, sem_c1,
                  sem_s0, sem_s1, sem_id0, sem_id1, sem_is0, sem_is1):
    c = lax.axis_index("c")
    s = lax.axis_index("s")
    g = c * NS + s

    idx_d = (idx_d0, idx_d1)
    idx_s = (idx_s0, idx_s1)
    buf_a = (buf_a0, buf_a1)
    buf_b = (buf_b0, buf_b1)
    buf_c = (buf_c0, buf_c1)
    sem_a = (sem_a0, sem_a1)
    sem_b = (sem_b0, sem_b1)
    sem_c = (sem_c0, sem_c1)
    sem_s = (sem_s0, sem_s1)
    sem_id = (sem_id0, sem_id1)
    sem_is = (sem_is0, sem_is1)

    def _load_idx(b, q):
        pltpu.async_copy(dst_hbm.at[g, b], idx_d[q], sem_id[q])
        pltpu.async_copy(src_hbm.at[g, b], idx_s[q], sem_is[q])

    def _wait_idx(q):
        pltpu.make_async_copy(dst_hbm.at[0, 0], idx_d[q], sem_id[q]).wait()
        pltpu.make_async_copy(src_hbm.at[0, 0], idx_s[q], sem_is[q]).wait()

    _load_idx(0, 0)

    zero = jnp.zeros((LANES,), jnp.float32)

    def _zrow(r, _):
        for j in range(H // LANES):
            buf_a0[r, pl.ds(j * LANES, LANES)] = zero
        return 0
    lax.fori_loop(0, K, _zrow, 0, unroll=False)

    rbase = pl.multiple_of(s * RPT, K)
    for j in range(RPT // K):
        pltpu.sync_copy(buf_a0, shared.at[pl.ds(rbase + j * K, K)])

    plsc.subcore_barrier()

    def _gather_ab(blk, j, p, q):
        pltpu.async_copy(a_hbm.at[idx_d[q].at[j]], buf_a[p], sem_a[p])
        pltpu.async_copy(b_hbm.at[idx_s[q].at[j]], buf_b[p], sem_b[p])

    def _load_c(blk, j, p):
        eoff = pl.multiple_of(g * EPT + (blk * IBLK + j) * K, 8)
        pltpu.async_copy(c_hbm.at[pl.ds(eoff, K)], buf_c[p], sem_c[p])

    def _step(blk, j, p, q):
        pltpu.make_async_copy(a_hbm.at[pl.ds(0, K)], buf_a[p], sem_a[p]).wait()
        pltpu.make_async_copy(b_hbm.at[pl.ds(0, K)], buf_b[p], sem_b[p]).wait()
        pltpu.make_async_copy(c_hbm.at[pl.ds(0, K)], buf_c[p], sem_c[p]).wait()

        def _row(r, _):
            for jj in range(H // LANES):
                sl = pl.ds(jj * LANES, LANES)
                v = buf_a[p][r, sl] + buf_b[p][r, sl] + buf_c[p][r, sl]
                buf_c[p][r, sl] = jnp.maximum(v, 0.0)
            return 0
        lax.fori_loop(0, K, _row, 0, unroll=False)

        pltpu.async_copy(buf_c[p], shared.at[idx_d[q].at[j]], sem_s[p],
                         add=True)
        nxt = j + 2

        @pl.when(nxt < IBLK)
        def _():
            _gather_ab(blk, nxt, p, q)

        pltpu.make_async_copy(buf_c[p], shared.at[pl.ds(0, K)], sem_s[p]).wait()

        @pl.when(nxt < IBLK)
        def _():
            _load_c(blk, nxt, p)

    for blk in range(NBLK):
        q = blk % 2
        _wait_idx(q)
        if blk + 1 < NBLK:
            _load_idx(blk + 1, 1 - q)
        _gather_ab(blk, 0, 0, q)
        _load_c(blk, 0, 0)
        _gather_ab(blk, 1, 1, q)
        _load_c(blk, 1, 1)

        def _super(t, _):
            _step(blk, 2 * t, 0, q)
            _step(blk, 2 * t + 1, 1, q)
            return 0
        lax.fori_loop(0, IBLK // 2, _super, 0, unroll=False)
        if IBLK % 2:
            _step(blk, IBLK - 1, 0, q)

    plsc.subcore_barrier()

    obase = pl.multiple_of(c * NP + rbase, K)
    for j in range(RPT // K):
        pltpu.sync_copy(shared.at[pl.ds(rbase + j * K, K)], buf_a0)
        pltpu.sync_copy(buf_a0, out_hbm.at[pl.ds(obase + j * K, K)])


@functools.cache
def _sc_edge_kernel():
    return pl.kernel(
        _sc_edge_body,
        out_type=jax.ShapeDtypeStruct((NC * NP, H), jnp.float32),
        mesh=plsc.VectorSubcoreMesh(core_axis_name="c", subcore_axis_name="s"),
        scratch_types=[
            pltpu.VMEM_SHARED((NP, H), jnp.float32),
        ] + [pltpu.VMEM((IBLK, K), jnp.int32)] * 4
          + [pltpu.VMEM((K, H), jnp.float32)] * 6
          + [pltpu.SemaphoreType.DMA] * 12,
    )


def _sc_edge(a, b, c, src, dst):
    src4 = src.reshape(NW, NBLK, IBLK, K)
    dst4 = dst.reshape(NW, NBLK, IBLK, K)
    return _sc_edge_kernel()(a, b, c, src4, dst4)


@jax.jit
def kernel(x, edge_index, edge_attr, W_msg0, b_msg0, W_upd0, b_upd0,
           W_msg1, b_msg1, W_upd1, b_upd1):
    src = edge_index[0]
    dst = edge_index[1]

    b0 = b_msg0.reshape(1, H)
    b1 = b_msg1.reshape(1, H)
    bu0 = b_upd0.reshape(1, H)
    bu1 = b_upd1.reshape(1, H)

    c0, c1 = _tc_cpre(edge_attr, W_msg0[2 * D:], b0, W_msg1[2 * H:], b1)

    a0, bmat0 = _tc_pre(x, W_msg0[:D], W_msg0[D:2 * D])
    aggs0 = _sc_edge(a0, bmat0, c0, src, dst).reshape(NC, NP, H)
    h, a1, bmat1 = _tc_update_fused(
        x, aggs0, W_upd0[:D], W_upd0[D:], bu0,
        W_msg1[:H], W_msg1[H:2 * H])

    aggs1 = _sc_edge(a1, bmat1, c1, src, dst).reshape(NC, NP, H)
    out = _tc_update(h, aggs1, W_upd1[:H], W_upd1[H:], bu1)
    return out


# speedup vs baseline: 5.5098x; 1.0534x over previous
import functools

import jax
import jax.numpy as jnp
from jax import lax
from jax.experimental import pallas as pl
from jax.experimental.pallas import tpu as pltpu
from jax.experimental.pallas import tpu_sc as plsc

N = 10000
E = 320000
D = 128
H = 128
ED = 16

NC = 2
NS = 16
NW = NC * NS
K = 40
EPT = E // NW
CHUNKS = EPT // K
IBLK = 25
NBLK = CHUNKS // IBLK
NP = 10240
RPT = NP // NS
LANES = 16

_DOT = functools.partial(
    lax.dot_general,
    dimension_numbers=(((1,), (0,)), ((), ())),
    preferred_element_type=jnp.float32,
    precision=lax.Precision.HIGHEST,
)


def _pre_body(x_ref, wi_ref, wj_ref, a_ref, b_ref):
    xb = x_ref[...]
    a_ref[...] = _DOT(xb, wi_ref[...])
    b_ref[...] = _DOT(xb, wj_ref[...])


def _cpre_body(ea_ref, we_ref, b_ref, c_ref):
    c_ref[...] = _DOT(ea_ref[...], we_ref[...]) + b_ref[...]


def _upd_fused_body(x_ref, a0_ref, a1_ref, wux_ref, wua_ref, bu_ref,
                    wi_ref, wj_ref, h_ref, a_ref, b_ref):
    agg = a0_ref[0] + a1_ref[0]
    h = _DOT(x_ref[...], wux_ref[...]) + _DOT(agg, wua_ref[...]) + bu_ref[...]
    h = jnp.maximum(h, 0.0)
    h_ref[...] = h
    a_ref[...] = _DOT(h, wi_ref[...])
    b_ref[...] = _DOT(h, wj_ref[...])


def _upd_body(x_ref, a0_ref, a1_ref, wux_ref, wua_ref, bu_ref, h_ref):
    agg = a0_ref[0] + a1_ref[0]
    h = _DOT(x_ref[...], wux_ref[...]) + _DOT(agg, wua_ref[...]) + bu_ref[...]
    h_ref[...] = jnp.maximum(h, 0.0)


_BN = 1000
_BE = 4000


def _node_spec(shape):
    return pl.BlockSpec((_BN,) + shape[1:], lambda i: (i,) + (0,) * (len(shape) - 1))


def _full_spec(shape):
    return pl.BlockSpec(shape, lambda i: (0,) * len(shape))


def _tc_pre(x, wi, wj):
    return pl.pallas_call(
        _pre_body,
        grid=(N // _BN,),
        in_specs=[_node_spec((N, D)), _full_spec((D, H)), _full_spec((D, H))],
        out_specs=[_node_spec((N, H)), _node_spec((N, H))],
        out_shape=[jax.ShapeDtypeStruct((N, H), jnp.float32)] * 2,
    )(x, wi, wj)


def _tc_cpre(ea, we, b):
    espec = pl.BlockSpec((_BE, ED), lambda i: (i, 0))
    ospec = pl.BlockSpec((_BE, H), lambda i: (i, 0))
    return pl.pallas_call(
        _cpre_body,
        grid=(E // _BE,),
        in_specs=[espec, _full_spec((ED, H)), _full_spec((1, H))],
        out_specs=ospec,
        out_shape=jax.ShapeDtypeStruct((E, H), jnp.float32),
    )(ea, we, b)


_A0SPEC = pl.BlockSpec((1, _BN, H), lambda i: (0, i, 0))
_A1SPEC = pl.BlockSpec((1, _BN, H), lambda i: (1, i, 0))


def _tc_update_fused(x, aggs, wux, wua, bu, wi, wj):
    return pl.pallas_call(
        _upd_fused_body,
        grid=(N // _BN,),
        in_specs=[_node_spec((N, D)), _A0SPEC, _A1SPEC,
                  _full_spec((D, H)), _full_spec((H, H)), _full_spec((1, H)),
                  _full_spec((H, H)), _full_spec((H, H))],
        out_specs=[_node_spec((N, H))] * 3,
        out_shape=[jax.ShapeDtypeStruct((N, H), jnp.float32)] * 3,
    )(x, aggs, aggs, wux, wua, bu, wi, wj)


def _tc_update(x, aggs, wux, wua, bu):
    return pl.pallas_call(
        _upd_body,
        grid=(N // _BN,),
        in_specs=[_node_spec((N, H)), _A0SPEC, _A1SPEC,
                  _full_spec((H, H)), _full_spec((H, H)), _full_spec((1, H))],
        out_specs=_node_spec((N, H)),
        out_shape=jax.ShapeDtypeStruct((N, H), jnp.float32),
    )(x, aggs, aggs, wux, wua, bu)


def _sc_edge_body(a_hbm, b_hbm, c_hbm, src_hbm, dst_hbm, out_hbm,
                  shared, idx_d0, idx_d1, idx_s0, idx_s1,
                  buf_a0, buf_a1, buf_b0, buf_b1, buf_c0, buf_c1,
                  sem_a0, sem_a1, sem_b0, sem_b1, sem_c0, sem_c1,
                  sem_s0, sem_s1, sem_id0, sem_id1, sem_is0, sem_is1):
    c = lax.axis_index("c")
    s = lax.axis_index("s")
    g = c * NS + s

    idx_d = (idx_d0, idx_d1)
    idx_s = (idx_s0, idx_s1)
    buf_a = (buf_a0, buf_a1)
    buf_b = (buf_b0, buf_b1)
    buf_c = (buf_c0, buf_c1)
    sem_a = (sem_a0, sem_a1)
    sem_b = (sem_b0, sem_b1)
    sem_c = (sem_c0, sem_c1)
    sem_s = (sem_s0, sem_s1)
    sem_id = (sem_id0, sem_id1)
    sem_is = (sem_is0, sem_is1)

    def _load_idx(b, q):
        pltpu.async_copy(dst_hbm.at[g, b], idx_d[q], sem_id[q])
        pltpu.async_copy(src_hbm.at[g, b], idx_s[q], sem_is[q])

    def _wait_idx(q):
        pltpu.make_async_copy(dst_hbm.at[0, 0], idx_d[q], sem_id[q]).wait()
        pltpu.make_async_copy(src_hbm.at[0, 0], idx_s[q], sem_is[q]).wait()

    _load_idx(0, 0)

    zero = jnp.zeros((LANES,), jnp.float32)

    def _zrow(r, _):
        for j in range(H // LANES):
            buf_a0[r, pl.ds(j * LANES, LANES)] = zero
        return 0
    lax.fori_loop(0, K, _zrow, 0, unroll=False)

    rbase = pl.multiple_of(s * RPT, K)
    for j in range(RPT // K):
        pltpu.sync_copy(buf_a0, shared.at[pl.ds(rbase + j * K, K)])

    plsc.subcore_barrier()

    def _gather_ab(blk, j, p, q):
        pltpu.async_copy(a_hbm.at[idx_d[q].at[j]], buf_a[p], sem_a[p])
        pltpu.async_copy(b_hbm.at[idx_s[q].at[j]], buf_b[p], sem_b[p])

    def _load_c(blk, j, p):
        eoff = pl.multiple_of(g * EPT + (blk * IBLK + j) * K, 8)
        pltpu.async_copy(c_hbm.at[pl.ds(eoff, K)], buf_c[p], sem_c[p])

    def _step(blk, j, p, q):
        pltpu.make_async_copy(a_hbm.at[pl.ds(0, K)], buf_a[p], sem_a[p]).wait()
        pltpu.make_async_copy(b_hbm.at[pl.ds(0, K)], buf_b[p], sem_b[p]).wait()
        pltpu.make_async_copy(c_hbm.at[pl.ds(0, K)], buf_c[p], sem_c[p]).wait()

        def _row(r, _):
            for jj in range(H // LANES):
                sl = pl.ds(jj * LANES, LANES)
                v = buf_a[p][r, sl] + buf_b[p][r, sl] + buf_c[p][r, sl]
                buf_c[p][r, sl] = jnp.maximum(v, 0.0)
            return 0
        lax.fori_loop(0, K, _row, 0, unroll=False)

        pltpu.async_copy(buf_c[p], shared.at[idx_d[q].at[j]], sem_s[p],
                         add=True)
        nxt = j + 2

        @pl.when(nxt < IBLK)
        def _():
            _gather_ab(blk, nxt, p, q)

        pltpu.make_async_copy(buf_c[p], shared.at[pl.ds(0, K)], sem_s[p]).wait()

        @pl.when(nxt < IBLK)
        def _():
            _load_c(blk, nxt, p)

    for blk in range(NBLK):
        q = blk % 2
        _wait_idx(q)
        if blk + 1 < NBLK:
            _load_idx(blk + 1, 1 - q)
        _gather_ab(blk, 0, 0, q)
        _load_c(blk, 0, 0)
        _gather_ab(blk, 1, 1, q)
        _load_c(blk, 1, 1)

        def _super(t, _):
            _step(blk, 2 * t, 0, q)
            _step(blk, 2 * t + 1, 1, q)
            return 0
        lax.fori_loop(0, IBLK // 2, _super, 0, unroll=False)
        if IBLK % 2:
            _step(blk, IBLK - 1, 0, q)

    plsc.subcore_barrier()

    obase = pl.multiple_of(c * NP + rbase, K)
    for j in range(RPT // K):
        pltpu.sync_copy(shared.at[pl.ds(rbase + j * K, K)], buf_a0)
        pltpu.sync_copy(buf_a0, out_hbm.at[pl.ds(obase + j * K, K)])


@functools.cache
def _sc_edge_kernel():
    return pl.kernel(
        _sc_edge_body,
        out_type=jax.ShapeDtypeStruct((NC * NP, H), jnp.float32),
        mesh=plsc.VectorSubcoreMesh(core_axis_name="c", subcore_axis_name="s"),
        scratch_types=[
            pltpu.VMEM_SHARED((NP, H), jnp.float32),
        ] + [pltpu.VMEM((IBLK, K), jnp.int32)] * 4
          + [pltpu.VMEM((K, H), jnp.float32)] * 6
          + [pltpu.SemaphoreType.DMA] * 12,
    )


def _sc_edge(a, b, c, src, dst):
    src4 = src.reshape(NW, NBLK, IBLK, K)
    dst4 = dst.reshape(NW, NBLK, IBLK, K)
    return _sc_edge_kernel()(a, b, c, src4, dst4)


@jax.jit
def kernel(x, edge_index, edge_attr, W_msg0, b_msg0, W_upd0, b_upd0,
           W_msg1, b_msg1, W_upd1, b_upd1):
    src = edge_index[0]
    dst = edge_index[1]

    b0 = b_msg0.reshape(1, H)
    b1 = b_msg1.reshape(1, H)
    bu0 = b_upd0.reshape(1, H)
    bu1 = b_upd1.reshape(1, H)

    c0 = _tc_cpre(edge_attr, W_msg0[2 * D:], b0)
    a0, bmat0 = _tc_pre(x, W_msg0[:D], W_msg0[D:2 * D])
    aggs0 = _sc_edge(a0, bmat0, c0, src, dst).reshape(NC, NP, H)
    c1 = _tc_cpre(edge_attr, W_msg1[2 * H:], b1)
    h, a1, bmat1 = _tc_update_fused(
        x, aggs0, W_upd0[:D], W_upd0[D:], bu0,
        W_msg1[:H], W_msg1[H:2 * H])

    aggs1 = _sc_edge(a1, bmat1, c1, src, dst).reshape(NC, NP, H)
    out = _tc_update(h, aggs1, W_upd1[:H], W_upd1[H:], bu1)
    return out
